# initial kernel scaffold (unmeasured)
import functools

import jax
import jax.numpy as jnp
from jax import lax
from jax.experimental import pallas as pl
from jax.experimental.pallas import tpu as pltpu

N_DEV = 16
SQ = 2048
DM = 1024
HQ = 8
DH = 128
NG = 4
GS = SQ // NG
NO = 8
CHUNK = SQ // N_DEV
SCALE = 0.08838834764831843

f32 = jnp.float32
bf16 = jnp.bfloat16


def _body(xg_ref, wq_ref, kgt_ref, vg_ref, wo_ref, out_ref,
          acc_ref, ctxp_ref, rs_ref,
          rs_send_sems, rs_recv_sems, ag_send_sems, ag_recv_sems):
    me = lax.axis_index("i")
    right = lax.rem(me + 1, N_DEV)

    for r in range(NG):
        q_r = jnp.dot(xg_ref[r], wq_ref[...],
                      preferred_element_type=f32)
        q_r = q_r.astype(bf16)
        for h in range(HQ):
            qh = q_r[:, h * DH:(h + 1) * DH]
            s = jnp.dot(qh, kgt_ref[r, h],
                        preferred_element_type=f32) * SCALE
            m = jnp.max(s, axis=1, keepdims=True)
            w = jnp.exp(s - m)
            w = w / jnp.sum(w, axis=1, keepdims=True)
            c = jnp.dot(w.astype(bf16), vg_ref[r, h],
                        preferred_element_type=f32)
            ctxp_ref[r, :, h * DH:(h + 1) * DH] = c.astype(bf16)
        p_r = jnp.dot(ctxp_ref[r], wo_ref[...],
                      preferred_element_type=f32)
        for o in range(NO):
            acc_ref[(o * NG + r) * 64:(o * NG + r + 1) * 64, :] = \
                p_r[o * 64:(o + 1) * 64, :]

    for s in range(N_DEV - 1):
        c_send = lax.rem(me - s + N_DEV, N_DEV)
        rdma = pltpu.make_async_remote_copy(
            src_ref=acc_ref.at[pl.ds(c_send * CHUNK, CHUNK), :],
            dst_ref=rs_ref.at[s],
            send_sem=rs_send_sems.at[s],
            recv_sem=rs_recv_sems.at[s],
            device_id=(right,),
            device_id_type=pl.DeviceIdType.MESH,
        )
        rdma.start()
        rdma.wait()
        c_recv = lax.rem(me - s - 1 + N_DEV, N_DEV)
        sl = pl.ds(c_recv * CHUNK, CHUNK)
        acc_ref[sl, :] = acc_ref[sl, :] + rs_ref[s]

    c_mine = lax.rem(me + 1, N_DEV)
    sl = pl.ds(c_mine * CHUNK, CHUNK)
    out_ref[sl, :] = acc_ref[sl, :]

    for t in range(N_DEV - 1):
        c = lax.rem(me + 1 - t + N_DEV, N_DEV)
        sl = pl.ds(c * CHUNK, CHUNK)
        rdma = pltpu.make_async_remote_copy(
            src_ref=out_ref.at[sl, :],
            dst_ref=out_ref.at[sl, :],
            send_sem=ag_send_sems.at[t],
            recv_sem=ag_recv_sems.at[t],
            device_id=(right,),
            device_id_type=pl.DeviceIdType.MESH,
        )
        rdma.start()
        rdma.wait()


def kernel(x, Wq, K_ext, V_ext, Wo):
    me = lax.axis_index("i")

    wq_my = lax.dynamic_slice(Wq, (0, me * DM), (DM, DM)).astype(bf16)
    wo_my = lax.dynamic_slice(Wo, (me * DM, 0), (DM, DM)).astype(bf16)

    xg = (x[0].reshape(NO, NG, 64, DM).transpose(1, 0, 2, 3)
          .reshape(NG, GS, DM).astype(bf16))
    kg = (K_ext[0].reshape(NO, NG, 64, HQ, DH).transpose(1, 3, 0, 2, 4)
          .reshape(NG, HQ, GS, DH).astype(bf16))
    kgt = kg.transpose(0, 1, 3, 2)
    vg = (V_ext[0].reshape(NO, NG, 64, HQ, DH).transpose(1, 3, 0, 2, 4)
          .reshape(NG, HQ, GS, DH).astype(bf16))

    out = pl.pallas_call(
        _body,
        out_shape=jax.ShapeDtypeStruct((SQ, DM), f32),
        in_specs=[pl.BlockSpec(memory_space=pltpu.VMEM)] * 5,
        out_specs=pl.BlockSpec(memory_space=pltpu.VMEM),
        scratch_shapes=[
            pltpu.VMEM((SQ, DM), f32),
            pltpu.VMEM((NG, GS, DM), bf16),
            pltpu.VMEM((N_DEV - 1, CHUNK, DM), f32),
            pltpu.SemaphoreType.DMA((N_DEV - 1,)),
            pltpu.SemaphoreType.DMA((N_DEV - 1,)),
            pltpu.SemaphoreType.DMA((N_DEV - 1,)),
            pltpu.SemaphoreType.DMA((N_DEV - 1,)),
        ],
        compiler_params=pltpu.CompilerParams(collective_id=0),
    )(xg, wq_my, kgt, vg, wo_my)
    return out.reshape(1, SQ, DM)


# baseline (device time: 283290 ns/iter reference)
import functools

import jax
import jax.numpy as jnp
from jax import lax
from jax.experimental import pallas as pl
from jax.experimental.pallas import tpu as pltpu

N_DEV = 16
SQ = 2048
DM = 1024
HQ = 8
DH = 128
NG = 4
GS = SQ // NG
NO = 8
CHUNK = SQ // N_DEV
SCALE = 0.08838834764831843

f32 = jnp.float32
bf16 = jnp.bfloat16


def _body(xg_ref, wq_ref, kgt_ref, vg_ref, wo_ref, out_ref,
          acc_ref, ctxp_ref, rs_ref,
          rs_send_sems, rs_recv_sems, ag_send_sems, ag_recv_sems):
    me = lax.axis_index("i")
    right = lax.rem(me + 1, N_DEV)

    for r in range(NG):
        q_r = jnp.dot(xg_ref[r], wq_ref[...],
                      preferred_element_type=f32)
        q_r = q_r.astype(bf16)
        for h in range(HQ):
            qh = q_r[:, h * DH:(h + 1) * DH]
            s = jnp.dot(qh, kgt_ref[r, h],
                        preferred_element_type=f32) * SCALE
            m = jnp.max(s, axis=1, keepdims=True)
            w = jnp.exp(s - m)
            w = w / jnp.sum(w, axis=1, keepdims=True)
            c = jnp.dot(w.astype(bf16), vg_ref[r, h],
                        preferred_element_type=f32)
            ctxp_ref[r, :, h * DH:(h + 1) * DH] = c.astype(bf16)
        p_r = jnp.dot(ctxp_ref[r], wo_ref[...],
                      preferred_element_type=f32)
        for o in range(NO):
            acc_ref[(o * NG + r) * 64:(o * NG + r + 1) * 64, :] = \
                p_r[o * 64:(o + 1) * 64, :]

    for s in range(N_DEV - 1):
        c_send = lax.rem(me - s + N_DEV, N_DEV)
        rdma = pltpu.make_async_remote_copy(
            src_ref=acc_ref.at[pl.ds(c_send * CHUNK, CHUNK), :],
            dst_ref=rs_ref.at[s],
            send_sem=rs_send_sems.at[s],
            recv_sem=rs_recv_sems.at[s],
            device_id=(right,),
            device_id_type=pl.DeviceIdType.MESH,
        )
        rdma.start()
        rdma.wait()
        c_recv = lax.rem(me - s - 1 + N_DEV, N_DEV)
        sl = pl.ds(c_recv * CHUNK, CHUNK)
        acc_ref[sl, :] = acc_ref[sl, :] + rs_ref[s]

    c_mine = lax.rem(me + 1, N_DEV)
    sl = pl.ds(c_mine * CHUNK, CHUNK)
    out_ref[sl, :] = acc_ref[sl, :]

    for t in range(N_DEV - 1):
        c = lax.rem(me + 1 - t + N_DEV, N_DEV)
        sl = pl.ds(c * CHUNK, CHUNK)
        rdma = pltpu.make_async_remote_copy(
            src_ref=out_ref.at[sl, :],
            dst_ref=out_ref.at[sl, :],
            send_sem=ag_send_sems.at[t],
            recv_sem=ag_recv_sems.at[t],
            device_id=(right,),
            device_id_type=pl.DeviceIdType.MESH,
        )
        rdma.start()
        rdma.wait()


def kernel(x, Wq, K_ext, V_ext, Wo):
    me = lax.axis_index("i")

    wq_my = lax.dynamic_slice(Wq, (0, me * DM), (DM, DM)).astype(bf16)
    wo_my = lax.dynamic_slice(Wo, (me * DM, 0), (DM, DM)).astype(bf16)

    xg = (x[0].reshape(NO, NG, 64, DM).transpose(1, 0, 2, 3)
          .reshape(NG, GS, DM).astype(bf16))
    kg = (K_ext[0].reshape(NO, NG, 64, HQ, DH).transpose(1, 3, 0, 2, 4)
          .reshape(NG, HQ, GS, DH).astype(bf16))
    kgt = kg.transpose(0, 1, 3, 2)
    vg = (V_ext[0].reshape(NO, NG, 64, HQ, DH).transpose(1, 3, 0, 2, 4)
          .reshape(NG, HQ, GS, DH).astype(bf16))

    out = pl.pallas_call(
        _body,
        out_shape=jax.ShapeDtypeStruct((SQ, DM), f32),
        in_specs=[pl.BlockSpec(memory_space=pltpu.VMEM)] * 5,
        out_specs=pl.BlockSpec(memory_space=pltpu.VMEM),
        scratch_shapes=[
            pltpu.VMEM((SQ, DM), f32),
            pltpu.VMEM((NG, GS, DM), bf16),
            pltpu.VMEM((N_DEV - 1, CHUNK, DM), f32),
            pltpu.SemaphoreType.DMA((N_DEV - 1,)),
            pltpu.SemaphoreType.DMA((N_DEV - 1,)),
            pltpu.SemaphoreType.DMA((N_DEV - 1,)),
            pltpu.SemaphoreType.DMA((N_DEV - 1,)),
        ],
    )(xg, wq_my, kgt, vg, wo_my)
    return out.reshape(1, SQ, DM)


# device time: 175418 ns/iter; 1.6149x vs baseline; 1.6149x over previous
import functools

import jax
import jax.numpy as jnp
from jax import lax
from jax.experimental import pallas as pl
from jax.experimental.pallas import tpu as pltpu

N_DEV = 16
SQ = 2048
DM = 1024
HQ = 8
DH = 128
NG = 4
GS = SQ // NG
NO = 8
CHUNK = SQ // N_DEV
SCALE = 0.08838834764831843

f32 = jnp.float32
bf16 = jnp.bfloat16


def _body(xg_ref, wq_ref, kgt_ref, vg_ref, wo_ref, out_ref,
          acc_ref, ctxp_ref, outg_ref, stage_ref,
          rs1_ref, rs2_ref, rs3_ref, rs4_ref,
          rs_send_sems, rs_recv_sems, ag_send_sems, ag_recv_sems):
    me = lax.axis_index("i")

    for r in range(NG):
        q_r = jnp.dot(xg_ref[r], wq_ref[...],
                      preferred_element_type=f32)
        q_r = q_r.astype(bf16)
        for h in range(HQ):
            qh = q_r[:, h * DH:(h + 1) * DH]
            s = jnp.dot(qh, kgt_ref[r, h],
                        preferred_element_type=f32) * SCALE
            m = jnp.max(s, axis=1, keepdims=True)
            w = jnp.exp(s - m)
            w = w / jnp.sum(w, axis=1, keepdims=True)
            c = jnp.dot(w.astype(bf16), vg_ref[r, h],
                        preferred_element_type=f32)
            ctxp_ref[r, :, h * DH:(h + 1) * DH] = c.astype(bf16)
        p_r = jnp.dot(ctxp_ref[r], wo_ref[...],
                      preferred_element_type=f32)
        for o in range(NO):
            acc_ref[(o * NG + r) * 64:(o * NG + r + 1) * 64, :] = \
                p_r[o * 64:(o + 1) * 64, :]

    tb = [
        (me ^ (me >> 1)) & 1,
        (me >> 1) & 1,
        (me >> 2) & 1,
        (me >> 3) & 1,
    ]
    partners = [me ^ 1, me ^ 3, me ^ 4, me ^ 8]
    rs_refs = [rs1_ref, rs2_ref, rs3_ref, rs4_ref]

    base = jnp.int32(0)
    size = SQ
    for p in range(4):
        half = size // 2
        keep_start = base + tb[p] * half
        send_start = base + (1 - tb[p]) * half
        stage_ref[:half, :] = acc_ref[pl.ds(send_start, half), :].astype(bf16)
        rdma = pltpu.make_async_remote_copy(
            src_ref=stage_ref.at[:half, :],
            dst_ref=rs_refs[p],
            send_sem=rs_send_sems.at[p],
            recv_sem=rs_recv_sems.at[p],
            device_id=(partners[p],),
            device_id_type=pl.DeviceIdType.MESH,
        )
        rdma.start()
        rdma.wait()
        sl = pl.ds(keep_start, half)
        acc_ref[sl, :] = acc_ref[sl, :] + rs_refs[p][...].astype(f32)
        base = keep_start
        size = half

    outg_ref[pl.ds(base, CHUNK), :] = acc_ref[pl.ds(base, CHUNK), :].astype(bf16)

    obase = base
    osize = CHUNK
    for j in range(4):
        p = 3 - j
        sl = pl.ds(obase, osize)
        rdma = pltpu.make_async_remote_copy(
            src_ref=outg_ref.at[sl, :],
            dst_ref=outg_ref.at[sl, :],
            send_sem=ag_send_sems.at[j],
            recv_sem=ag_recv_sems.at[j],
            device_id=(partners[p],),
            device_id_type=pl.DeviceIdType.MESH,
        )
        rdma.start()
        rdma.wait()
        obase = obase - tb[p] * osize
        osize *= 2

    out_ref[...] = outg_ref[...].astype(f32)


def kernel(x, Wq, K_ext, V_ext, Wo):
    me = lax.axis_index("i")

    wq_my = lax.dynamic_slice(Wq, (0, me * DM), (DM, DM)).astype(bf16)
    wo_my = lax.dynamic_slice(Wo, (me * DM, 0), (DM, DM)).astype(bf16)

    xg = (x[0].reshape(NO, NG, 64, DM).transpose(1, 0, 2, 3)
          .reshape(NG, GS, DM).astype(bf16))
    kg = (K_ext[0].reshape(NO, NG, 64, HQ, DH).transpose(1, 3, 0, 2, 4)
          .reshape(NG, HQ, GS, DH).astype(bf16))
    kgt = kg.transpose(0, 1, 3, 2)
    vg = (V_ext[0].reshape(NO, NG, 64, HQ, DH).transpose(1, 3, 0, 2, 4)
          .reshape(NG, HQ, GS, DH).astype(bf16))

    out = pl.pallas_call(
        _body,
        out_shape=jax.ShapeDtypeStruct((SQ, DM), f32),
        in_specs=[pl.BlockSpec(memory_space=pltpu.VMEM)] * 5,
        out_specs=pl.BlockSpec(memory_space=pltpu.VMEM),
        scratch_shapes=[
            pltpu.VMEM((SQ, DM), f32),
            pltpu.VMEM((NG, GS, DM), bf16),
            pltpu.VMEM((SQ, DM), bf16),
            pltpu.VMEM((SQ // 2, DM), bf16),
            pltpu.VMEM((SQ // 2, DM), bf16),
            pltpu.VMEM((SQ // 4, DM), bf16),
            pltpu.VMEM((SQ // 8, DM), bf16),
            pltpu.VMEM((SQ // 16, DM), bf16),
            pltpu.SemaphoreType.DMA((4,)),
            pltpu.SemaphoreType.DMA((4,)),
            pltpu.SemaphoreType.DMA((4,)),
            pltpu.SemaphoreType.DMA((4,)),
        ],
        compiler_params=pltpu.CompilerParams(
            vmem_limit_bytes=60 * 1024 * 1024,
        ),
    )(xg, wq_my, kgt, vg, wo_my)
    return out.reshape(1, SQ, DM)
